# baseline (speedup 1.0000x reference)
import functools

import numpy as np
import jax
import jax.numpy as jnp
from jax import lax
from jax.experimental import pallas as pl
from jax.experimental.pallas import tpu as pltpu
from jax.experimental.pallas import tpu_sc as plsc

_N = 10000
_NP = 10240
_E = 160000
_F_IN = 128
_EMB = 16
_HID = 32
_VOCAB = 12
_K = 33

_NW = 32
_NT = 16
_CH = 128
_BE = 640


def _expansion_consts(out_ch):
    s = np.zeros((32, _K * out_ch), np.float32)
    for k in range(32):
        s[k, k * out_ch:(k + 1) * out_ch] = 1.0
    c = np.zeros((1, _K * out_ch), np.float32)
    c[0, 32 * out_ch:] = 1.0
    r = np.zeros((_K * out_ch, 128), np.float32)
    for k in range(_K):
        for o in range(out_ch):
            r[k * out_ch + o, o] = 1.0
    return s, c, r


_S1_np, _C1_np, _R1_np = _expansion_consts(_EMB)
_S2_np, _C2_np, _R2_np = _expansion_consts(_HID)
_E16_np = np.zeros((1, 128), np.float32)
_E16_np[0, 16] = 1.0


def _sc_gather(table, idx, d):
    e = idx.shape[0]
    n_chunks = e // _CH
    iters = (n_chunks + _NW - 1) // _NW
    mesh = plsc.VectorSubcoreMesh(core_axis_name="c", subcore_axis_name="s")

    @functools.partial(
        pl.kernel,
        out_type=jax.ShapeDtypeStruct((e, d), jnp.float32),
        mesh=mesh,
        scratch_types=[
            pltpu.VMEM((_CH,), jnp.int32),
            pltpu.VMEM((_CH, d), jnp.float32),
            pltpu.SemaphoreType.DMA,
        ],
    )
    def gk(table_hbm, idx_hbm, out_hbm, idx_v, rows_v, sem):
        wid = lax.axis_index("s") * 2 + lax.axis_index("c")

        def body(i, carry):
            c = i * _NW + wid

            @pl.when(c < n_chunks)
            def _():
                base = c * _CH
                pltpu.sync_copy(idx_hbm.at[pl.ds(base, _CH)], idx_v)
                pltpu.async_copy(table_hbm.at[idx_v], rows_v, sem).wait()
                pltpu.sync_copy(rows_v, out_hbm.at[pl.ds(base, _CH)])

            return carry

        lax.fori_loop(0, iters, body, 0)

    return gk(table, idx)


def _sc_scatter_add(vals, idx, zeros_init):
    e = vals.shape[0]
    n_chunks = e // _CH
    iters = (n_chunks + _NW - 1) // _NW
    rpt = _NP // _NT
    mesh = plsc.VectorSubcoreMesh(core_axis_name="c", subcore_axis_name="s")

    @functools.partial(
        pl.kernel,
        out_type=jax.ShapeDtypeStruct((2 * _NP, 128), jnp.float32),
        mesh=mesh,
        scratch_types=[
            pltpu.VMEM((_CH,), jnp.int32),
            pltpu.VMEM((_CH, 128), jnp.float32),
            pltpu.VMEM_SHARED((_NP, 128), jnp.float32),
            pltpu.SemaphoreType.DMA,
        ],
    )
    def sk(vals_hbm, idx_hbm, zeros_hbm, out_hbm, idx_v, rows_v, acc_sh, sem):
        cid = lax.axis_index("c")
        sid = lax.axis_index("s")
        wid = sid * 2 + cid

        pltpu.sync_copy(zeros_hbm.at[pl.ds(sid * rpt, rpt)],
                        acc_sh.at[pl.ds(sid * rpt, rpt)])
        plsc.subcore_barrier()

        def body(i, carry):
            c = i * _NW + wid

            @pl.when(c < n_chunks)
            def _():
                base = c * _CH
                pltpu.sync_copy(idx_hbm.at[pl.ds(base, _CH)], idx_v)
                pltpu.sync_copy(vals_hbm.at[pl.ds(base, _CH)], rows_v)
                pltpu.sync_copy(rows_v, acc_sh.at[idx_v], add=True)

            return carry

        lax.fori_loop(0, iters, body, 0)
        plsc.subcore_barrier()
        pltpu.sync_copy(acc_sh.at[pl.ds(sid * rpt, rpt)],
                        out_hbm.at[pl.ds(cid * _NP + sid * rpt, rpt)])

    return sk(vals, idx, zeros_init)


def _edge1_body(xj_ref, ea_ref, w1a_ref, b1a_ref, w1p_ref, s1_ref, c1_ref,
                r1_ref, e16_ref, out_ref):
    h = jnp.maximum(
        jnp.dot(ea_ref[...], w1a_ref[...],
                preferred_element_type=jnp.float32, precision=lax.Precision.HIGHEST) + b1a_ref[...], 0.0)
    t = jnp.dot(xj_ref[...], w1p_ref[...], preferred_element_type=jnp.float32, precision=lax.Precision.HIGHEST)
    hh = jnp.dot(h, s1_ref[...], preferred_element_type=jnp.float32, precision=lax.Precision.HIGHEST) + c1_ref[...]
    out_ref[...] = jnp.dot(t * hh, r1_ref[...],
                           preferred_element_type=jnp.float32, precision=lax.Precision.HIGHEST) + e16_ref[...]


def _edge2_body(hj_ref, ea_ref, w2a_ref, b2a_ref, w2p_ref, s2_ref, c2_ref,
                r2_ref, out_ref):
    h = jnp.maximum(
        jnp.dot(ea_ref[...], w2a_ref[...],
                preferred_element_type=jnp.float32, precision=lax.Precision.HIGHEST) + b2a_ref[...], 0.0)
    t = jnp.dot(hj_ref[:, 0:_EMB], w2p_ref[...],
                preferred_element_type=jnp.float32, precision=lax.Precision.HIGHEST)
    hh = jnp.dot(h, s2_ref[...], preferred_element_type=jnp.float32, precision=lax.Precision.HIGHEST) + c2_ref[...]
    out_ref[...] = jnp.dot(t * hh, r2_ref[...],
                           preferred_element_type=jnp.float32, precision=lax.Precision.HIGHEST)


def _edge_combine1(xj, ea, w1a, b1a, w1p, s1, c1, r1, e16):
    grid = (_E // _BE,)
    return pl.pallas_call(
        _edge1_body,
        grid=grid,
        in_specs=[
            pl.BlockSpec((_BE, _F_IN), lambda i: (i, 0)),
            pl.BlockSpec((_BE, 16), lambda i: (i, 0)),
            pl.BlockSpec((16, 32), lambda i: (0, 0)),
            pl.BlockSpec((1, 32), lambda i: (0, 0)),
            pl.BlockSpec((_F_IN, _K * _EMB), lambda i: (0, 0)),
            pl.BlockSpec((32, _K * _EMB), lambda i: (0, 0)),
            pl.BlockSpec((1, _K * _EMB), lambda i: (0, 0)),
            pl.BlockSpec((_K * _EMB, 128), lambda i: (0, 0)),
            pl.BlockSpec((1, 128), lambda i: (0, 0)),
        ],
        out_specs=pl.BlockSpec((_BE, 128), lambda i: (i, 0)),
        out_shape=jax.ShapeDtypeStruct((_E, 128), jnp.float32),
    )(xj, ea, w1a, b1a, w1p, s1, c1, r1, e16)


def _edge_combine2(hj, ea, w2a, b2a, w2p, s2, c2, r2):
    grid = (_E // _BE,)
    return pl.pallas_call(
        _edge2_body,
        grid=grid,
        in_specs=[
            pl.BlockSpec((_BE, _F_IN), lambda i: (i, 0)),
            pl.BlockSpec((_BE, 16), lambda i: (i, 0)),
            pl.BlockSpec((16, 32), lambda i: (0, 0)),
            pl.BlockSpec((1, 32), lambda i: (0, 0)),
            pl.BlockSpec((_EMB, _K * _HID), lambda i: (0, 0)),
            pl.BlockSpec((32, _K * _HID), lambda i: (0, 0)),
            pl.BlockSpec((1, _K * _HID), lambda i: (0, 0)),
            pl.BlockSpec((_K * _HID, 128), lambda i: (0, 0)),
        ],
        out_specs=pl.BlockSpec((_BE, 128), lambda i: (i, 0)),
        out_shape=jax.ShapeDtypeStruct((_E, 128), jnp.float32),
    )(hj, ea, w2a, b2a, w2p, s2, c2, r2)


def _node1_body(x_ref, p_ref, w1_ref, rb1_ref, hid_ref, deg_ref):
    acc = p_ref[0:_N, 0:32] + p_ref[_NP:_NP + _N, 0:32]
    deg = jnp.maximum(acc[:, 16:17], 1.0)
    mean = acc[:, 0:_EMB] / deg
    h = jnp.maximum(
        jnp.dot(x_ref[...], w1_ref[...],
                preferred_element_type=jnp.float32, precision=lax.Precision.HIGHEST) + mean + rb1_ref[...], 0.0)
    hid_ref[...] = jnp.concatenate(
        [h, jnp.zeros((_N, _F_IN - _EMB), jnp.float32)], axis=1)
    deg_ref[...] = deg


def _node_update1(x, p1, w1, rb1):
    return pl.pallas_call(
        _node1_body,
        out_shape=(jax.ShapeDtypeStruct((_N, _F_IN), jnp.float32),
                   jax.ShapeDtypeStruct((_N, 1), jnp.float32)),
    )(x, p1, w1, rb1)


def _final_body(hid_ref, p_ref, deg_ref, w2_ref, rb2_ref, wl_ref, bl_ref,
                out_ref):
    acc = p_ref[0:_N, 0:32] + p_ref[_NP:_NP + _N, 0:32]
    mean = acc / deg_ref[...]
    h = jnp.maximum(
        jnp.dot(hid_ref[:, 0:_EMB], w2_ref[...],
                preferred_element_type=jnp.float32, precision=lax.Precision.HIGHEST) + mean + rb2_ref[...], 0.0)
    pooled = jnp.sum(h, axis=0, keepdims=True) * (1.0 / _N)
    logits = jnp.dot(pooled, wl_ref[...],
                     preferred_element_type=jnp.float32, precision=lax.Precision.HIGHEST) + bl_ref[...]
    m = jnp.max(logits, axis=1, keepdims=True)
    lse = jnp.log(jnp.sum(jnp.exp(logits - m), axis=1, keepdims=True)) + m
    out_ref[...] = logits - lse


def _final(hid, p2, deg, w2, rb2, wl, bl):
    return pl.pallas_call(
        _final_body,
        out_shape=jax.ShapeDtypeStruct((1, _VOCAB), jnp.float32),
    )(hid, p2, deg, w2, rb2, wl, bl)


def kernel(x, edge_index, edge_attr, w1a, b1a, w1b, b1b, W1, rb1,
           w2a, b2a, w2b, b2b, W2, rb2, Wl, bl):
    src = edge_index[0]
    dst = edge_index[1]

    w1p = jnp.concatenate(
        [w1b.reshape(32, _F_IN, _EMB).transpose(1, 0, 2).reshape(_F_IN, 32 * _EMB),
         b1b.reshape(_F_IN, _EMB)], axis=1)
    w2p = jnp.concatenate(
        [w2b.reshape(32, _EMB, _HID).transpose(1, 0, 2).reshape(_EMB, 32 * _HID),
         b2b.reshape(_EMB, _HID)], axis=1)

    s1 = jnp.asarray(_S1_np)
    c1 = jnp.asarray(_C1_np)
    r1 = jnp.asarray(_R1_np)
    s2 = jnp.asarray(_S2_np)
    c2 = jnp.asarray(_C2_np)
    r2 = jnp.asarray(_R2_np)
    e16 = jnp.asarray(_E16_np)
    zeros_init = jnp.zeros((_NP, 128), jnp.float32)

    xj = _sc_gather(x, src, _F_IN)
    msg1 = _edge_combine1(xj, edge_attr, w1a, b1a.reshape(1, 32),
                          w1p, s1, c1, r1, e16)
    p1 = _sc_scatter_add(msg1, dst, zeros_init)
    hid, deg = _node_update1(x, p1, W1, rb1.reshape(1, _EMB))
    hj = _sc_gather(hid, src, _F_IN)
    msg2 = _edge_combine2(hj, edge_attr, w2a, b2a.reshape(1, 32),
                          w2p, s2, c2, r2)
    p2 = _sc_scatter_add(msg2, dst, zeros_init)
    return _final(hid, p2, deg, W2, rb2.reshape(1, _HID), Wl,
                  bl.reshape(1, _VOCAB))


# speedup vs baseline: 3.3992x; 3.3992x over previous
import functools

import numpy as np
import jax
import jax.numpy as jnp
from jax import lax
from jax.experimental import pallas as pl
from jax.experimental.pallas import tpu as pltpu
from jax.experimental.pallas import tpu_sc as plsc

_N = 10000
_NP = 10240
_E = 160000
_F_IN = 128
_EMB = 16
_HID = 32
_VOCAB = 12
_K = 33

_NW = 32
_NT = 16
_CH = 128
_BE = 640


def _expansion_consts(out_ch):
    s = np.zeros((32, _K * out_ch), np.float32)
    for k in range(32):
        s[k, k * out_ch:(k + 1) * out_ch] = 1.0
    c = np.zeros((1, _K * out_ch), np.float32)
    c[0, 32 * out_ch:] = 1.0
    r = np.zeros((_K * out_ch, 128), np.float32)
    for k in range(_K):
        for o in range(out_ch):
            r[k * out_ch + o, o] = 1.0
    return s, c, r


_S1_np, _C1_np, _R1_np = _expansion_consts(_EMB)
_S2_np, _C2_np, _R2_np = _expansion_consts(_HID)
_E16_np = np.zeros((1, 128), np.float32)
_E16_np[0, 16] = 1.0


def _sc_gather(table, idx, d):
    e = idx.shape[0]
    n_chunks = e // _CH
    iters = (n_chunks + _NW - 1) // _NW
    mesh = plsc.VectorSubcoreMesh(core_axis_name="c", subcore_axis_name="s")

    @functools.partial(
        pl.kernel,
        out_type=jax.ShapeDtypeStruct((e, d), jnp.float32),
        mesh=mesh,
        scratch_types=[
            pltpu.VMEM((_CH,), jnp.int32),
            pltpu.VMEM((_CH, d), jnp.float32),
            pltpu.SemaphoreType.DMA,
        ],
    )
    def gk(table_hbm, idx_hbm, out_hbm, idx_v, rows_v, sem):
        wid = lax.axis_index("s") * 2 + lax.axis_index("c")

        def body(i, carry):
            c = i * _NW + wid

            @pl.when(c < n_chunks)
            def _():
                base = c * _CH
                pltpu.sync_copy(idx_hbm.at[pl.ds(base, _CH)], idx_v)
                pltpu.async_copy(table_hbm.at[idx_v], rows_v, sem).wait()
                pltpu.sync_copy(rows_v, out_hbm.at[pl.ds(base, _CH)])

            return carry

        lax.fori_loop(0, iters, body, 0)

    return gk(table, idx)


def _sc_scatter_add(vals, idx, zeros_init):
    e = vals.shape[0]
    n_chunks = e // _CH
    iters = (n_chunks + _NW - 1) // _NW
    rpt = _NP // _NT
    mesh = plsc.VectorSubcoreMesh(core_axis_name="c", subcore_axis_name="s")

    @functools.partial(
        pl.kernel,
        out_type=jax.ShapeDtypeStruct((2 * _NP, 128), jnp.float32),
        mesh=mesh,
        scratch_types=[
            pltpu.VMEM((_CH,), jnp.int32),
            pltpu.VMEM((_CH, 128), jnp.float32),
            pltpu.VMEM_SHARED((_NP, 128), jnp.float32),
            pltpu.SemaphoreType.DMA,
        ],
    )
    def sk(vals_hbm, idx_hbm, zeros_hbm, out_hbm, idx_v, rows_v, acc_sh, sem):
        cid = lax.axis_index("c")
        sid = lax.axis_index("s")
        wid = sid * 2 + cid

        pltpu.sync_copy(zeros_hbm.at[pl.ds(sid * rpt, rpt)],
                        acc_sh.at[pl.ds(sid * rpt, rpt)])
        plsc.subcore_barrier()

        def body(i, carry):
            c = i * _NW + wid

            @pl.when(c < n_chunks)
            def _():
                base = c * _CH
                pltpu.sync_copy(idx_hbm.at[pl.ds(base, _CH)], idx_v)
                pltpu.sync_copy(vals_hbm.at[pl.ds(base, _CH)], rows_v)
                pltpu.sync_copy(rows_v, acc_sh.at[idx_v], add=True)

            return carry

        lax.fori_loop(0, iters, body, 0)
        plsc.subcore_barrier()
        pltpu.sync_copy(acc_sh.at[pl.ds(sid * rpt, rpt)],
                        out_hbm.at[pl.ds(cid * _NP + sid * rpt, rpt)])

    return sk(vals, idx, zeros_init)


def _edge1_body(xj_ref, ea_ref, w1a_ref, b1a_ref, w1p_ref, s1_ref, c1_ref,
                r1_ref, e16_ref, out_ref):
    h = jnp.maximum(
        jnp.dot(ea_ref[...], w1a_ref[...],
                preferred_element_type=jnp.float32) + b1a_ref[...], 0.0)
    t = jnp.dot(xj_ref[...], w1p_ref[...], preferred_element_type=jnp.float32)
    hh = jnp.dot(h, s1_ref[...], preferred_element_type=jnp.float32) + c1_ref[...]
    out_ref[...] = jnp.dot(t * hh, r1_ref[...],
                           preferred_element_type=jnp.float32) + e16_ref[...]


def _edge2_body(hj_ref, ea_ref, w2a_ref, b2a_ref, w2p_ref, s2_ref, c2_ref,
                r2_ref, out_ref):
    h = jnp.maximum(
        jnp.dot(ea_ref[...], w2a_ref[...],
                preferred_element_type=jnp.float32) + b2a_ref[...], 0.0)
    t = jnp.dot(hj_ref[:, 0:_EMB], w2p_ref[...],
                preferred_element_type=jnp.float32)
    hh = jnp.dot(h, s2_ref[...], preferred_element_type=jnp.float32) + c2_ref[...]
    out_ref[...] = jnp.dot(t * hh, r2_ref[...],
                           preferred_element_type=jnp.float32)


def _edge_combine1(xj, ea, w1a, b1a, w1p, s1, c1, r1, e16):
    grid = (_E // _BE,)
    return pl.pallas_call(
        _edge1_body,
        grid=grid,
        in_specs=[
            pl.BlockSpec((_BE, _F_IN), lambda i: (i, 0)),
            pl.BlockSpec((_BE, 16), lambda i: (i, 0)),
            pl.BlockSpec((16, 32), lambda i: (0, 0)),
            pl.BlockSpec((1, 32), lambda i: (0, 0)),
            pl.BlockSpec((_F_IN, _K * _EMB), lambda i: (0, 0)),
            pl.BlockSpec((32, _K * _EMB), lambda i: (0, 0)),
            pl.BlockSpec((1, _K * _EMB), lambda i: (0, 0)),
            pl.BlockSpec((_K * _EMB, 128), lambda i: (0, 0)),
            pl.BlockSpec((1, 128), lambda i: (0, 0)),
        ],
        out_specs=pl.BlockSpec((_BE, 128), lambda i: (i, 0)),
        out_shape=jax.ShapeDtypeStruct((_E, 128), jnp.float32),
    )(xj, ea, w1a, b1a, w1p, s1, c1, r1, e16)


def _edge_combine2(hj, ea, w2a, b2a, w2p, s2, c2, r2):
    grid = (_E // _BE,)
    return pl.pallas_call(
        _edge2_body,
        grid=grid,
        in_specs=[
            pl.BlockSpec((_BE, _F_IN), lambda i: (i, 0)),
            pl.BlockSpec((_BE, 16), lambda i: (i, 0)),
            pl.BlockSpec((16, 32), lambda i: (0, 0)),
            pl.BlockSpec((1, 32), lambda i: (0, 0)),
            pl.BlockSpec((_EMB, _K * _HID), lambda i: (0, 0)),
            pl.BlockSpec((32, _K * _HID), lambda i: (0, 0)),
            pl.BlockSpec((1, _K * _HID), lambda i: (0, 0)),
            pl.BlockSpec((_K * _HID, 128), lambda i: (0, 0)),
        ],
        out_specs=pl.BlockSpec((_BE, 128), lambda i: (i, 0)),
        out_shape=jax.ShapeDtypeStruct((_E, 128), jnp.float32),
    )(hj, ea, w2a, b2a, w2p, s2, c2, r2)


def _node1_body(x_ref, p_ref, w1_ref, rb1_ref, hid_ref, deg_ref):
    acc = p_ref[0:_N, 0:32] + p_ref[_NP:_NP + _N, 0:32]
    deg = jnp.maximum(acc[:, 16:17], 1.0)
    mean = acc[:, 0:_EMB] / deg
    h = jnp.maximum(
        jnp.dot(x_ref[...], w1_ref[...],
                preferred_element_type=jnp.float32) + mean + rb1_ref[...], 0.0)
    hid_ref[...] = jnp.concatenate(
        [h, jnp.zeros((_N, _F_IN - _EMB), jnp.float32)], axis=1)
    deg_ref[...] = deg


def _node_update1(x, p1, w1, rb1):
    return pl.pallas_call(
        _node1_body,
        out_shape=(jax.ShapeDtypeStruct((_N, _F_IN), jnp.float32),
                   jax.ShapeDtypeStruct((_N, 1), jnp.float32)),
    )(x, p1, w1, rb1)


def _final_body(hid_ref, p_ref, deg_ref, w2_ref, rb2_ref, wl_ref, bl_ref,
                out_ref):
    acc = p_ref[0:_N, 0:32] + p_ref[_NP:_NP + _N, 0:32]
    mean = acc / deg_ref[...]
    h = jnp.maximum(
        jnp.dot(hid_ref[:, 0:_EMB], w2_ref[...],
                preferred_element_type=jnp.float32) + mean + rb2_ref[...], 0.0)
    pooled = jnp.sum(h, axis=0, keepdims=True) * (1.0 / _N)
    logits = jnp.dot(pooled, wl_ref[...],
                     preferred_element_type=jnp.float32) + bl_ref[...]
    m = jnp.max(logits, axis=1, keepdims=True)
    lse = jnp.log(jnp.sum(jnp.exp(logits - m), axis=1, keepdims=True)) + m
    out_ref[...] = logits - lse


def _final(hid, p2, deg, w2, rb2, wl, bl):
    return pl.pallas_call(
        _final_body,
        out_shape=jax.ShapeDtypeStruct((1, _VOCAB), jnp.float32),
    )(hid, p2, deg, w2, rb2, wl, bl)


def kernel(x, edge_index, edge_attr, w1a, b1a, w1b, b1b, W1, rb1,
           w2a, b2a, w2b, b2b, W2, rb2, Wl, bl):
    src = edge_index[0]
    dst = edge_index[1]

    w1p = jnp.concatenate(
        [w1b.reshape(32, _F_IN, _EMB).transpose(1, 0, 2).reshape(_F_IN, 32 * _EMB),
         b1b.reshape(_F_IN, _EMB)], axis=1)
    w2p = jnp.concatenate(
        [w2b.reshape(32, _EMB, _HID).transpose(1, 0, 2).reshape(_EMB, 32 * _HID),
         b2b.reshape(_EMB, _HID)], axis=1)

    s1 = jnp.asarray(_S1_np)
    c1 = jnp.asarray(_C1_np)
    r1 = jnp.asarray(_R1_np)
    s2 = jnp.asarray(_S2_np)
    c2 = jnp.asarray(_C2_np)
    r2 = jnp.asarray(_R2_np)
    e16 = jnp.asarray(_E16_np)
    zeros_init = jnp.zeros((_NP, 128), jnp.float32)

    xj = _sc_gather(x, src, _F_IN)
    msg1 = _edge_combine1(xj, edge_attr, w1a, b1a.reshape(1, 32),
                          w1p, s1, c1, r1, e16)
    p1 = _sc_scatter_add(msg1, dst, zeros_init)
    hid, deg = _node_update1(x, p1, W1, rb1.reshape(1, _EMB))
    hj = _sc_gather(hid, src, _F_IN)
    msg2 = _edge_combine2(hj, edge_attr, w2a, b2a.reshape(1, 32),
                          w2p, s2, c2, r2)
    p2 = _sc_scatter_add(msg2, dst, zeros_init)
    return _final(hid, p2, deg, W2, rb2.reshape(1, _HID), Wl,
                  bl.reshape(1, _VOCAB))


# speedup vs baseline: 3.7759x; 1.1108x over previous
import functools

import numpy as np
import jax
import jax.numpy as jnp
from jax import lax
from jax.experimental import pallas as pl
from jax.experimental.pallas import tpu as pltpu
from jax.experimental.pallas import tpu_sc as plsc

_N = 10000
_NP = 10240
_E = 160000
_F_IN = 128
_EMB = 16
_HID = 32
_VOCAB = 12
_K = 33

_NW = 32
_NT = 16
_CH = 128
_BE = 640


def _expansion_consts(out_ch):
    s = np.zeros((32, _K * out_ch), np.float32)
    for k in range(32):
        s[k, k * out_ch:(k + 1) * out_ch] = 1.0
    c = np.zeros((1, _K * out_ch), np.float32)
    c[0, 32 * out_ch:] = 1.0
    r = np.zeros((_K * out_ch, 128), np.float32)
    for k in range(_K):
        for o in range(out_ch):
            r[k * out_ch + o, o] = 1.0
    return s, c, r


_S1_np, _C1_np, _R1_np = _expansion_consts(_EMB)
_S2_np, _C2_np, _R2_np = _expansion_consts(_HID)
_E16_np = np.zeros((1, 128), np.float32)
_E16_np[0, 16] = 1.0


_NBUF = 4
_NBUF_SC = 2


def _sc_gather(table, idx, d):
    e = idx.shape[0]
    n_chunks = e // _CH
    iters = (n_chunks + _NW - 1) // _NW
    full = n_chunks // _NW
    ss = full // _NBUF
    mesh = plsc.VectorSubcoreMesh(core_axis_name="c", subcore_axis_name="s")

    @functools.partial(
        pl.kernel,
        out_type=jax.ShapeDtypeStruct((e, d), jnp.float32),
        mesh=mesh,
        scratch_types=[
            pltpu.VMEM((_NBUF, _CH), jnp.int32),
            pltpu.VMEM((_NBUF, _CH, d), jnp.float32),
            pltpu.SemaphoreType.DMA,
            pltpu.SemaphoreType.DMA,
            pltpu.SemaphoreType.DMA,
        ],
    )
    def gk(table_hbm, idx_hbm, out_hbm, idx_b, rows_b, isem, gsem, wsem):
        wid = lax.axis_index("s") * 2 + lax.axis_index("c")

        def superstep(s, carry):
            i0 = s * _NBUF

            def base(j):
                return ((i0 + j) * _NW + wid) * _CH

            for j in range(_NBUF):
                pltpu.async_copy(idx_hbm.at[pl.ds(base(j), _CH)],
                                 idx_b.at[j], isem)
            for j in range(_NBUF):
                pltpu.make_async_copy(idx_hbm.at[pl.ds(base(j), _CH)],
                                      idx_b.at[j], isem).wait()
            for j in range(_NBUF):
                pltpu.async_copy(table_hbm.at[idx_b.at[j]], rows_b.at[j],
                                 gsem)
            for j in range(_NBUF):
                pltpu.make_async_copy(table_hbm.at[idx_b.at[j]],
                                      rows_b.at[j], gsem).wait()
            for j in range(_NBUF):
                pltpu.async_copy(rows_b.at[j],
                                 out_hbm.at[pl.ds(base(j), _CH)], wsem)
            for j in range(_NBUF):
                pltpu.make_async_copy(rows_b.at[j],
                                      out_hbm.at[pl.ds(base(j), _CH)],
                                      wsem).wait()
            return carry

        lax.fori_loop(0, ss, superstep, 0)

        def tail(i, carry):
            c = i * _NW + wid

            @pl.when(c < n_chunks)
            def _():
                b = c * _CH
                pltpu.sync_copy(idx_hbm.at[pl.ds(b, _CH)], idx_b.at[0])
                pltpu.async_copy(table_hbm.at[idx_b.at[0]], rows_b.at[0],
                                 gsem).wait()
                pltpu.sync_copy(rows_b.at[0], out_hbm.at[pl.ds(b, _CH)])

            return carry

        lax.fori_loop(ss * _NBUF, iters, tail, 0)

    return gk(table, idx)


def _sc_scatter_add(vals, idx, zeros_init):
    e = vals.shape[0]
    n_chunks = e // _CH
    iters = (n_chunks + _NW - 1) // _NW
    rpt = _NP // _NT
    mesh = plsc.VectorSubcoreMesh(core_axis_name="c", subcore_axis_name="s")

    @functools.partial(
        pl.kernel,
        out_type=jax.ShapeDtypeStruct((2 * _NP, 128), jnp.float32),
        mesh=mesh,
        scratch_types=[
            pltpu.VMEM((_NBUF_SC, _CH), jnp.int32),
            pltpu.VMEM((_NBUF_SC, _CH, 128), jnp.float32),
            pltpu.VMEM_SHARED((_NP, 128), jnp.float32),
            pltpu.SemaphoreType.DMA,
            pltpu.SemaphoreType.DMA,
            pltpu.SemaphoreType.DMA,
        ],
    )
    def sk(vals_hbm, idx_hbm, zeros_hbm, out_hbm, idx_b, rows_b, acc_sh,
           isem, vsem, ssem):
        nbuf = _NBUF_SC
        cid = lax.axis_index("c")
        sid = lax.axis_index("s")
        wid = sid * 2 + cid

        pltpu.sync_copy(zeros_hbm.at[pl.ds(sid * rpt, rpt)],
                        acc_sh.at[pl.ds(sid * rpt, rpt)])
        plsc.subcore_barrier()

        full = n_chunks // _NW
        ss = full // nbuf

        def superstep(s, carry):
            i0 = s * nbuf

            def base(j):
                return ((i0 + j) * _NW + wid) * _CH

            for j in range(nbuf):
                pltpu.async_copy(idx_hbm.at[pl.ds(base(j), _CH)],
                                 idx_b.at[j], isem)
                pltpu.async_copy(vals_hbm.at[pl.ds(base(j), _CH)],
                                 rows_b.at[j], vsem)
            for j in range(nbuf):
                pltpu.make_async_copy(idx_hbm.at[pl.ds(base(j), _CH)],
                                      idx_b.at[j], isem).wait()
                pltpu.make_async_copy(vals_hbm.at[pl.ds(base(j), _CH)],
                                      rows_b.at[j], vsem).wait()
            for j in range(nbuf):
                pltpu.async_copy(rows_b.at[j], acc_sh.at[idx_b.at[j]],
                                 ssem, add=True)
            for j in range(nbuf):
                pltpu.make_async_copy(rows_b.at[j], acc_sh.at[idx_b.at[j]],
                                      ssem).wait()
            return carry

        lax.fori_loop(0, ss, superstep, 0)

        def tail(i, carry):
            c = i * _NW + wid

            @pl.when(c < n_chunks)
            def _():
                b = c * _CH
                pltpu.sync_copy(idx_hbm.at[pl.ds(b, _CH)], idx_b.at[0])
                pltpu.sync_copy(vals_hbm.at[pl.ds(b, _CH)], rows_b.at[0])
                pltpu.sync_copy(rows_b.at[0], acc_sh.at[idx_b.at[0]],
                                add=True)

            return carry

        lax.fori_loop(ss * nbuf, iters, tail, 0)
        plsc.subcore_barrier()
        pltpu.sync_copy(acc_sh.at[pl.ds(sid * rpt, rpt)],
                        out_hbm.at[pl.ds(cid * _NP + sid * rpt, rpt)])

    return sk(vals, idx, zeros_init)


def _edge1_body(xj_ref, ea_ref, w1a_ref, b1a_ref, w1p_ref, s1_ref, c1_ref,
                r1_ref, e16_ref, out_ref):
    h = jnp.maximum(
        jnp.dot(ea_ref[...], w1a_ref[...],
                preferred_element_type=jnp.float32) + b1a_ref[...], 0.0)
    t = jnp.dot(xj_ref[...], w1p_ref[...], preferred_element_type=jnp.float32)
    hh = jnp.dot(h, s1_ref[...], preferred_element_type=jnp.float32) + c1_ref[...]
    out_ref[...] = jnp.dot(t * hh, r1_ref[...],
                           preferred_element_type=jnp.float32) + e16_ref[...]


def _edge2_body(hj_ref, ea_ref, w2a_ref, b2a_ref, w2p_ref, s2_ref, c2_ref,
                r2_ref, out_ref):
    h = jnp.maximum(
        jnp.dot(ea_ref[...], w2a_ref[...],
                preferred_element_type=jnp.float32) + b2a_ref[...], 0.0)
    t = jnp.dot(hj_ref[:, 0:_EMB], w2p_ref[...],
                preferred_element_type=jnp.float32)
    hh = jnp.dot(h, s2_ref[...], preferred_element_type=jnp.float32) + c2_ref[...]
    out_ref[...] = jnp.dot(t * hh, r2_ref[...],
                           preferred_element_type=jnp.float32)


def _edge_combine1(xj, ea, w1a, b1a, w1p, s1, c1, r1, e16):
    grid = (_E // _BE,)
    return pl.pallas_call(
        _edge1_body,
        grid=grid,
        in_specs=[
            pl.BlockSpec((_BE, _F_IN), lambda i: (i, 0)),
            pl.BlockSpec((_BE, 16), lambda i: (i, 0)),
            pl.BlockSpec((16, 32), lambda i: (0, 0)),
            pl.BlockSpec((1, 32), lambda i: (0, 0)),
            pl.BlockSpec((_F_IN, _K * _EMB), lambda i: (0, 0)),
            pl.BlockSpec((32, _K * _EMB), lambda i: (0, 0)),
            pl.BlockSpec((1, _K * _EMB), lambda i: (0, 0)),
            pl.BlockSpec((_K * _EMB, 128), lambda i: (0, 0)),
            pl.BlockSpec((1, 128), lambda i: (0, 0)),
        ],
        out_specs=pl.BlockSpec((_BE, 128), lambda i: (i, 0)),
        out_shape=jax.ShapeDtypeStruct((_E, 128), jnp.float32),
    )(xj, ea, w1a, b1a, w1p, s1, c1, r1, e16)


def _edge_combine2(hj, ea, w2a, b2a, w2p, s2, c2, r2):
    grid = (_E // _BE,)
    return pl.pallas_call(
        _edge2_body,
        grid=grid,
        in_specs=[
            pl.BlockSpec((_BE, _F_IN), lambda i: (i, 0)),
            pl.BlockSpec((_BE, 16), lambda i: (i, 0)),
            pl.BlockSpec((16, 32), lambda i: (0, 0)),
            pl.BlockSpec((1, 32), lambda i: (0, 0)),
            pl.BlockSpec((_EMB, _K * _HID), lambda i: (0, 0)),
            pl.BlockSpec((32, _K * _HID), lambda i: (0, 0)),
            pl.BlockSpec((1, _K * _HID), lambda i: (0, 0)),
            pl.BlockSpec((_K * _HID, 128), lambda i: (0, 0)),
        ],
        out_specs=pl.BlockSpec((_BE, 128), lambda i: (i, 0)),
        out_shape=jax.ShapeDtypeStruct((_E, 128), jnp.float32),
    )(hj, ea, w2a, b2a, w2p, s2, c2, r2)


def _node1_body(x_ref, p_ref, w1_ref, rb1_ref, hid_ref, deg_ref):
    acc = p_ref[0:_N, 0:32] + p_ref[_NP:_NP + _N, 0:32]
    deg = jnp.maximum(acc[:, 16:17], 1.0)
    mean = acc[:, 0:_EMB] / deg
    h = jnp.maximum(
        jnp.dot(x_ref[...], w1_ref[...],
                preferred_element_type=jnp.float32) + mean + rb1_ref[...], 0.0)
    hid_ref[...] = jnp.concatenate(
        [h, jnp.zeros((_N, _F_IN - _EMB), jnp.float32)], axis=1)
    deg_ref[...] = deg


def _node_update1(x, p1, w1, rb1):
    return pl.pallas_call(
        _node1_body,
        out_shape=(jax.ShapeDtypeStruct((_N, _F_IN), jnp.float32),
                   jax.ShapeDtypeStruct((_N, 1), jnp.float32)),
    )(x, p1, w1, rb1)


def _final_body(hid_ref, p_ref, deg_ref, w2_ref, rb2_ref, wl_ref, bl_ref,
                out_ref):
    acc = p_ref[0:_N, 0:32] + p_ref[_NP:_NP + _N, 0:32]
    mean = acc / deg_ref[...]
    h = jnp.maximum(
        jnp.dot(hid_ref[:, 0:_EMB], w2_ref[...],
                preferred_element_type=jnp.float32) + mean + rb2_ref[...], 0.0)
    pooled = jnp.sum(h, axis=0, keepdims=True) * (1.0 / _N)
    logits = jnp.dot(pooled, wl_ref[...],
                     preferred_element_type=jnp.float32) + bl_ref[...]
    m = jnp.max(logits, axis=1, keepdims=True)
    lse = jnp.log(jnp.sum(jnp.exp(logits - m), axis=1, keepdims=True)) + m
    out_ref[...] = logits - lse


def _final(hid, p2, deg, w2, rb2, wl, bl):
    return pl.pallas_call(
        _final_body,
        out_shape=jax.ShapeDtypeStruct((1, _VOCAB), jnp.float32),
    )(hid, p2, deg, w2, rb2, wl, bl)


def kernel(x, edge_index, edge_attr, w1a, b1a, w1b, b1b, W1, rb1,
           w2a, b2a, w2b, b2b, W2, rb2, Wl, bl):
    src = edge_index[0]
    dst = edge_index[1]

    w1p = jnp.concatenate(
        [w1b.reshape(32, _F_IN, _EMB).transpose(1, 0, 2).reshape(_F_IN, 32 * _EMB),
         b1b.reshape(_F_IN, _EMB)], axis=1)
    w2p = jnp.concatenate(
        [w2b.reshape(32, _EMB, _HID).transpose(1, 0, 2).reshape(_EMB, 32 * _HID),
         b2b.reshape(_EMB, _HID)], axis=1)

    s1 = jnp.asarray(_S1_np)
    c1 = jnp.asarray(_C1_np)
    r1 = jnp.asarray(_R1_np)
    s2 = jnp.asarray(_S2_np)
    c2 = jnp.asarray(_C2_np)
    r2 = jnp.asarray(_R2_np)
    e16 = jnp.asarray(_E16_np)
    zeros_init = jnp.zeros((_NP, 128), jnp.float32)

    xj = _sc_gather(x, src, _F_IN)
    msg1 = _edge_combine1(xj, edge_attr, w1a, b1a.reshape(1, 32),
                          w1p, s1, c1, r1, e16)
    p1 = _sc_scatter_add(msg1, dst, zeros_init)
    hid, deg = _node_update1(x, p1, W1, rb1.reshape(1, _EMB))
    hj = _sc_gather(hid, src, _F_IN)
    msg2 = _edge_combine2(hj, edge_attr, w2a, b2a.reshape(1, 32),
                          w2p, s2, c2, r2)
    p2 = _sc_scatter_add(msg2, dst, zeros_init)
    return _final(hid, p2, deg, W2, rb2.reshape(1, _HID), Wl,
                  bl.reshape(1, _VOCAB))


# speedup vs baseline: 4.2373x; 1.1222x over previous
import functools

import numpy as np
import jax
import jax.numpy as jnp
from jax import lax
from jax.experimental import pallas as pl
from jax.experimental.pallas import tpu as pltpu
from jax.experimental.pallas import tpu_sc as plsc

_N = 10000
_NP = 10240
_E = 160000
_F_IN = 128
_EMB = 16
_HID = 32
_VOCAB = 12
_K = 33

_NW = 32
_NT = 16
_CH = 128
_BE = 1280


def _expansion_consts(out_ch):
    s = np.zeros((32, _K * out_ch), np.float32)
    for k in range(32):
        s[k, k * out_ch:(k + 1) * out_ch] = 1.0
    c = np.zeros((1, _K * out_ch), np.float32)
    c[0, 32 * out_ch:] = 1.0
    r = np.zeros((_K * out_ch, 128), np.float32)
    for k in range(_K):
        for o in range(out_ch):
            r[k * out_ch + o, o] = 1.0
    return s, c, r


_S1_np, _C1_np, _R1_np = _expansion_consts(_EMB)
_S2_np, _C2_np, _R2_np = _expansion_consts(_HID)
_E16_np = np.zeros((1, 128), np.float32)
_E16_np[0, 16] = 1.0


_NBUF = 4
_NBUF_SC = 2


def _sc_gather(table, idx, d):
    e = idx.shape[0]
    n_chunks = e // _CH
    iters = (n_chunks + _NW - 1) // _NW
    full = n_chunks // _NW
    ss = full // _NBUF
    mesh = plsc.VectorSubcoreMesh(core_axis_name="c", subcore_axis_name="s")

    @functools.partial(
        pl.kernel,
        out_type=jax.ShapeDtypeStruct((e, d), jnp.float32),
        mesh=mesh,
        scratch_types=[
            pltpu.VMEM((_NBUF, _CH), jnp.int32),
            pltpu.VMEM((_NBUF, _CH, d), jnp.float32),
            pltpu.SemaphoreType.DMA,
            pltpu.SemaphoreType.DMA,
            pltpu.SemaphoreType.DMA,
        ],
    )
    def gk(table_hbm, idx_hbm, out_hbm, idx_b, rows_b, isem, gsem, wsem):
        wid = lax.axis_index("s") * 2 + lax.axis_index("c")

        def superstep(s, carry):
            i0 = s * _NBUF

            def base(j):
                return ((i0 + j) * _NW + wid) * _CH

            for j in range(_NBUF):
                pltpu.async_copy(idx_hbm.at[pl.ds(base(j), _CH)],
                                 idx_b.at[j], isem)
            for j in range(_NBUF):
                pltpu.make_async_copy(idx_hbm.at[pl.ds(base(j), _CH)],
                                      idx_b.at[j], isem).wait()
            for j in range(_NBUF):
                pltpu.async_copy(table_hbm.at[idx_b.at[j]], rows_b.at[j],
                                 gsem)
            for j in range(_NBUF):
                pltpu.make_async_copy(table_hbm.at[idx_b.at[j]],
                                      rows_b.at[j], gsem).wait()
            for j in range(_NBUF):
                pltpu.async_copy(rows_b.at[j],
                                 out_hbm.at[pl.ds(base(j), _CH)], wsem)
            for j in range(_NBUF):
                pltpu.make_async_copy(rows_b.at[j],
                                      out_hbm.at[pl.ds(base(j), _CH)],
                                      wsem).wait()
            return carry

        lax.fori_loop(0, ss, superstep, 0)

        def tail(i, carry):
            c = i * _NW + wid

            @pl.when(c < n_chunks)
            def _():
                b = c * _CH
                pltpu.sync_copy(idx_hbm.at[pl.ds(b, _CH)], idx_b.at[0])
                pltpu.async_copy(table_hbm.at[idx_b.at[0]], rows_b.at[0],
                                 gsem).wait()
                pltpu.sync_copy(rows_b.at[0], out_hbm.at[pl.ds(b, _CH)])

            return carry

        lax.fori_loop(ss * _NBUF, iters, tail, 0)

    return gk(table, idx)


def _sc_scatter_add(vals, idx, zeros_init):
    e = vals.shape[0]
    n_chunks = e // _CH
    iters = (n_chunks + _NW - 1) // _NW
    rpt = _NP // _NT
    mesh = plsc.VectorSubcoreMesh(core_axis_name="c", subcore_axis_name="s")

    @functools.partial(
        pl.kernel,
        out_type=jax.ShapeDtypeStruct((2 * _NP, 128), jnp.float32),
        mesh=mesh,
        scratch_types=[
            pltpu.VMEM((_NBUF_SC, _CH), jnp.int32),
            pltpu.VMEM((_NBUF_SC, _CH, 128), jnp.float32),
            pltpu.VMEM_SHARED((_NP, 128), jnp.float32),
            pltpu.SemaphoreType.DMA,
            pltpu.SemaphoreType.DMA,
            pltpu.SemaphoreType.DMA,
        ],
    )
    def sk(vals_hbm, idx_hbm, zeros_hbm, out_hbm, idx_b, rows_b, acc_sh,
           isem, vsem, ssem):
        nbuf = _NBUF_SC
        cid = lax.axis_index("c")
        sid = lax.axis_index("s")
        wid = sid * 2 + cid

        pltpu.sync_copy(zeros_hbm.at[pl.ds(sid * rpt, rpt)],
                        acc_sh.at[pl.ds(sid * rpt, rpt)])
        plsc.subcore_barrier()

        full = n_chunks // _NW
        ss = full // nbuf

        def superstep(s, carry):
            i0 = s * nbuf

            def base(j):
                return ((i0 + j) * _NW + wid) * _CH

            for j in range(nbuf):
                pltpu.async_copy(idx_hbm.at[pl.ds(base(j), _CH)],
                                 idx_b.at[j], isem)
                pltpu.async_copy(vals_hbm.at[pl.ds(base(j), _CH)],
                                 rows_b.at[j], vsem)
            for j in range(nbuf):
                pltpu.make_async_copy(idx_hbm.at[pl.ds(base(j), _CH)],
                                      idx_b.at[j], isem).wait()
                pltpu.make_async_copy(vals_hbm.at[pl.ds(base(j), _CH)],
                                      rows_b.at[j], vsem).wait()
            for j in range(nbuf):
                pltpu.async_copy(rows_b.at[j], acc_sh.at[idx_b.at[j]],
                                 ssem, add=True)
            for j in range(nbuf):
                pltpu.make_async_copy(rows_b.at[j], acc_sh.at[idx_b.at[j]],
                                      ssem).wait()
            return carry

        lax.fori_loop(0, ss, superstep, 0)

        def tail(i, carry):
            c = i * _NW + wid

            @pl.when(c < n_chunks)
            def _():
                b = c * _CH
                pltpu.sync_copy(idx_hbm.at[pl.ds(b, _CH)], idx_b.at[0])
                pltpu.sync_copy(vals_hbm.at[pl.ds(b, _CH)], rows_b.at[0])
                pltpu.sync_copy(rows_b.at[0], acc_sh.at[idx_b.at[0]],
                                add=True)

            return carry

        lax.fori_loop(ss * nbuf, iters, tail, 0)
        plsc.subcore_barrier()
        pltpu.sync_copy(acc_sh.at[pl.ds(sid * rpt, rpt)],
                        out_hbm.at[pl.ds(cid * _NP + sid * rpt, rpt)])

    return sk(vals, idx, zeros_init)


def _edge1_body(xj_ref, ea_ref, w1a_ref, b1a_ref, w1p_ref, s1_ref, c1_ref,
                r1_ref, e16_ref, out_ref):
    h = jnp.maximum(
        jnp.dot(ea_ref[...], w1a_ref[...],
                preferred_element_type=jnp.float32) + b1a_ref[...], 0.0)
    t = jnp.dot(xj_ref[...], w1p_ref[...], preferred_element_type=jnp.float32)
    hh = jnp.dot(h, s1_ref[...], preferred_element_type=jnp.float32) + c1_ref[...]
    out_ref[...] = jnp.dot(t * hh, r1_ref[...],
                           preferred_element_type=jnp.float32) + e16_ref[...]


def _edge2_body(hj_ref, ea_ref, w2a_ref, b2a_ref, w2p_ref, s2_ref, c2_ref,
                r2_ref, out_ref):
    h = jnp.maximum(
        jnp.dot(ea_ref[...], w2a_ref[...],
                preferred_element_type=jnp.float32) + b2a_ref[...], 0.0)
    t = jnp.dot(hj_ref[:, 0:_EMB], w2p_ref[...],
                preferred_element_type=jnp.float32)
    hh = jnp.dot(h, s2_ref[...], preferred_element_type=jnp.float32) + c2_ref[...]
    out_ref[...] = jnp.dot(t * hh, r2_ref[...],
                           preferred_element_type=jnp.float32)


def _edge_combine1(xj, ea, w1a, b1a, w1p, s1, c1, r1, e16):
    grid = (_E // _BE,)
    return pl.pallas_call(
        _edge1_body,
        grid=grid,
        in_specs=[
            pl.BlockSpec((_BE, _F_IN), lambda i: (i, 0)),
            pl.BlockSpec((_BE, 16), lambda i: (i, 0)),
            pl.BlockSpec((16, 32), lambda i: (0, 0)),
            pl.BlockSpec((1, 32), lambda i: (0, 0)),
            pl.BlockSpec((_F_IN, _K * _EMB), lambda i: (0, 0)),
            pl.BlockSpec((32, _K * _EMB), lambda i: (0, 0)),
            pl.BlockSpec((1, _K * _EMB), lambda i: (0, 0)),
            pl.BlockSpec((_K * _EMB, 128), lambda i: (0, 0)),
            pl.BlockSpec((1, 128), lambda i: (0, 0)),
        ],
        out_specs=pl.BlockSpec((_BE, 128), lambda i: (i, 0)),
        out_shape=jax.ShapeDtypeStruct((_E, 128), jnp.float32),
    )(xj, ea, w1a, b1a, w1p, s1, c1, r1, e16)


def _edge_combine2(hj, ea, w2a, b2a, w2p, s2, c2, r2):
    grid = (_E // _BE,)
    return pl.pallas_call(
        _edge2_body,
        grid=grid,
        in_specs=[
            pl.BlockSpec((_BE, _F_IN), lambda i: (i, 0)),
            pl.BlockSpec((_BE, 16), lambda i: (i, 0)),
            pl.BlockSpec((16, 32), lambda i: (0, 0)),
            pl.BlockSpec((1, 32), lambda i: (0, 0)),
            pl.BlockSpec((_EMB, _K * _HID), lambda i: (0, 0)),
            pl.BlockSpec((32, _K * _HID), lambda i: (0, 0)),
            pl.BlockSpec((1, _K * _HID), lambda i: (0, 0)),
            pl.BlockSpec((_K * _HID, 128), lambda i: (0, 0)),
        ],
        out_specs=pl.BlockSpec((_BE, 128), lambda i: (i, 0)),
        out_shape=jax.ShapeDtypeStruct((_E, 128), jnp.float32),
    )(hj, ea, w2a, b2a, w2p, s2, c2, r2)


def _node1_body(x_ref, p_ref, w1_ref, rb1_ref, hid_ref, deg_ref):
    acc = p_ref[0:_N, 0:32] + p_ref[_NP:_NP + _N, 0:32]
    deg = jnp.maximum(acc[:, 16:17], 1.0)
    mean = acc[:, 0:_EMB] / deg
    h = jnp.maximum(
        jnp.dot(x_ref[...], w1_ref[...],
                preferred_element_type=jnp.float32) + mean + rb1_ref[...], 0.0)
    hid_ref[...] = jnp.concatenate(
        [h, jnp.zeros((_N, _F_IN - _EMB), jnp.float32)], axis=1)
    deg_ref[...] = deg


def _node_update1(x, p1, w1, rb1):
    return pl.pallas_call(
        _node1_body,
        out_shape=(jax.ShapeDtypeStruct((_N, _F_IN), jnp.float32),
                   jax.ShapeDtypeStruct((_N, 1), jnp.float32)),
    )(x, p1, w1, rb1)


def _final_body(hid_ref, p_ref, deg_ref, w2_ref, rb2_ref, wl_ref, bl_ref,
                out_ref):
    acc = p_ref[0:_N, 0:32] + p_ref[_NP:_NP + _N, 0:32]
    mean = acc / deg_ref[...]
    h = jnp.maximum(
        jnp.dot(hid_ref[:, 0:_EMB], w2_ref[...],
                preferred_element_type=jnp.float32) + mean + rb2_ref[...], 0.0)
    pooled = jnp.sum(h, axis=0, keepdims=True) * (1.0 / _N)
    logits = jnp.dot(pooled, wl_ref[...],
                     preferred_element_type=jnp.float32) + bl_ref[...]
    m = jnp.max(logits, axis=1, keepdims=True)
    lse = jnp.log(jnp.sum(jnp.exp(logits - m), axis=1, keepdims=True)) + m
    out_ref[...] = logits - lse


def _final(hid, p2, deg, w2, rb2, wl, bl):
    return pl.pallas_call(
        _final_body,
        out_shape=jax.ShapeDtypeStruct((1, _VOCAB), jnp.float32),
    )(hid, p2, deg, w2, rb2, wl, bl)


def kernel(x, edge_index, edge_attr, w1a, b1a, w1b, b1b, W1, rb1,
           w2a, b2a, w2b, b2b, W2, rb2, Wl, bl):
    src = edge_index[0]
    dst = edge_index[1]

    w1p = jnp.concatenate(
        [w1b.reshape(32, _F_IN, _EMB).transpose(1, 0, 2).reshape(_F_IN, 32 * _EMB),
         b1b.reshape(_F_IN, _EMB)], axis=1)
    w2p = jnp.concatenate(
        [w2b.reshape(32, _EMB, _HID).transpose(1, 0, 2).reshape(_EMB, 32 * _HID),
         b2b.reshape(_EMB, _HID)], axis=1)

    s1 = jnp.asarray(_S1_np)
    c1 = jnp.asarray(_C1_np)
    r1 = jnp.asarray(_R1_np)
    s2 = jnp.asarray(_S2_np)
    c2 = jnp.asarray(_C2_np)
    r2 = jnp.asarray(_R2_np)
    e16 = jnp.asarray(_E16_np)
    zeros_init = jnp.zeros((_NP, 128), jnp.float32)

    xj = _sc_gather(x, src, _F_IN)
    msg1 = _edge_combine1(xj, edge_attr, w1a, b1a.reshape(1, 32),
                          w1p, s1, c1, r1, e16)
    p1 = _sc_scatter_add(msg1, dst, zeros_init)
    hid, deg = _node_update1(x, p1, W1, rb1.reshape(1, _EMB))
    hj = _sc_gather(hid, src, _F_IN)
    msg2 = _edge_combine2(hj, edge_attr, w2a, b2a.reshape(1, 32),
                          w2p, s2, c2, r2)
    p2 = _sc_scatter_add(msg2, dst, zeros_init)
    return _final(hid, p2, deg, W2, rb2.reshape(1, _HID), Wl,
                  bl.reshape(1, _VOCAB))


# speedup vs baseline: 4.2909x; 1.0126x over previous
import functools

import numpy as np
import jax
import jax.numpy as jnp
from jax import lax
from jax.experimental import pallas as pl
from jax.experimental.pallas import tpu as pltpu
from jax.experimental.pallas import tpu_sc as plsc

_N = 10000
_NP = 10240
_E = 160000
_F_IN = 128
_EMB = 16
_HID = 32
_VOCAB = 12
_K = 33

_NW = 32
_NT = 16
_CH = 128
_BE = 1600


def _expansion_consts(out_ch):
    s = np.zeros((32, _K * out_ch), np.float32)
    for k in range(32):
        s[k, k * out_ch:(k + 1) * out_ch] = 1.0
    c = np.zeros((1, _K * out_ch), np.float32)
    c[0, 32 * out_ch:] = 1.0
    r = np.zeros((_K * out_ch, 128), np.float32)
    for k in range(_K):
        for o in range(out_ch):
            r[k * out_ch + o, o] = 1.0
    return s, c, r


_S1_np, _C1_np, _R1_np = _expansion_consts(_EMB)
_S2_np, _C2_np, _R2_np = _expansion_consts(_HID)
_E16_np = np.zeros((1, 128), np.float32)
_E16_np[0, 16] = 1.0


_NBUF = 4
_NBUF_SC = 2


def _sc_gather(table, idx, d):
    e = idx.shape[0]
    n_chunks = e // _CH
    iters = (n_chunks + _NW - 1) // _NW
    full = n_chunks // _NW
    ss = full // _NBUF
    mesh = plsc.VectorSubcoreMesh(core_axis_name="c", subcore_axis_name="s")

    @functools.partial(
        pl.kernel,
        out_type=jax.ShapeDtypeStruct((e, d), jnp.float32),
        mesh=mesh,
        scratch_types=[
            pltpu.VMEM((_NBUF, _CH), jnp.int32),
            pltpu.VMEM((_NBUF, _CH, d), jnp.float32),
            pltpu.SemaphoreType.DMA,
            pltpu.SemaphoreType.DMA,
            pltpu.SemaphoreType.DMA,
        ],
    )
    def gk(table_hbm, idx_hbm, out_hbm, idx_b, rows_b, isem, gsem, wsem):
        wid = lax.axis_index("s") * 2 + lax.axis_index("c")

        def superstep(s, carry):
            i0 = s * _NBUF

            def base(j):
                return ((i0 + j) * _NW + wid) * _CH

            for j in range(_NBUF):
                pltpu.async_copy(idx_hbm.at[pl.ds(base(j), _CH)],
                                 idx_b.at[j], isem)
            for j in range(_NBUF):
                pltpu.make_async_copy(idx_hbm.at[pl.ds(base(j), _CH)],
                                      idx_b.at[j], isem).wait()
            for j in range(_NBUF):
                pltpu.async_copy(table_hbm.at[idx_b.at[j]], rows_b.at[j],
                                 gsem)
            for j in range(_NBUF):
                pltpu.make_async_copy(table_hbm.at[idx_b.at[j]],
                                      rows_b.at[j], gsem).wait()
            for j in range(_NBUF):
                pltpu.async_copy(rows_b.at[j],
                                 out_hbm.at[pl.ds(base(j), _CH)], wsem)
            for j in range(_NBUF):
                pltpu.make_async_copy(rows_b.at[j],
                                      out_hbm.at[pl.ds(base(j), _CH)],
                                      wsem).wait()
            return carry

        lax.fori_loop(0, ss, superstep, 0)

        def tail(i, carry):
            c = i * _NW + wid

            @pl.when(c < n_chunks)
            def _():
                b = c * _CH
                pltpu.sync_copy(idx_hbm.at[pl.ds(b, _CH)], idx_b.at[0])
                pltpu.async_copy(table_hbm.at[idx_b.at[0]], rows_b.at[0],
                                 gsem).wait()
                pltpu.sync_copy(rows_b.at[0], out_hbm.at[pl.ds(b, _CH)])

            return carry

        lax.fori_loop(ss * _NBUF, iters, tail, 0)

    return gk(table, idx)


def _sc_scatter_add(vals, idx, zeros_init):
    e = vals.shape[0]
    n_chunks = e // _CH
    iters = (n_chunks + _NW - 1) // _NW
    rpt = _NP // _NT
    mesh = plsc.VectorSubcoreMesh(core_axis_name="c", subcore_axis_name="s")

    @functools.partial(
        pl.kernel,
        out_type=jax.ShapeDtypeStruct((2 * _NP, 128), jnp.float32),
        mesh=mesh,
        scratch_types=[
            pltpu.VMEM((_NBUF_SC, _CH), jnp.int32),
            pltpu.VMEM((_NBUF_SC, _CH, 128), jnp.float32),
            pltpu.VMEM_SHARED((_NP, 128), jnp.float32),
            pltpu.SemaphoreType.DMA,
            pltpu.SemaphoreType.DMA,
            pltpu.SemaphoreType.DMA,
        ],
    )
    def sk(vals_hbm, idx_hbm, zeros_hbm, out_hbm, idx_b, rows_b, acc_sh,
           isem, vsem, ssem):
        nbuf = _NBUF_SC
        cid = lax.axis_index("c")
        sid = lax.axis_index("s")
        wid = sid * 2 + cid

        pltpu.sync_copy(zeros_hbm.at[pl.ds(sid * rpt, rpt)],
                        acc_sh.at[pl.ds(sid * rpt, rpt)])
        plsc.subcore_barrier()

        full = n_chunks // _NW
        ss = full // nbuf

        def superstep(s, carry):
            i0 = s * nbuf

            def base(j):
                return ((i0 + j) * _NW + wid) * _CH

            for j in range(nbuf):
                pltpu.async_copy(idx_hbm.at[pl.ds(base(j), _CH)],
                                 idx_b.at[j], isem)
                pltpu.async_copy(vals_hbm.at[pl.ds(base(j), _CH)],
                                 rows_b.at[j], vsem)
            for j in range(nbuf):
                pltpu.make_async_copy(idx_hbm.at[pl.ds(base(j), _CH)],
                                      idx_b.at[j], isem).wait()
                pltpu.make_async_copy(vals_hbm.at[pl.ds(base(j), _CH)],
                                      rows_b.at[j], vsem).wait()
            for j in range(nbuf):
                pltpu.async_copy(rows_b.at[j], acc_sh.at[idx_b.at[j]],
                                 ssem, add=True)
            for j in range(nbuf):
                pltpu.make_async_copy(rows_b.at[j], acc_sh.at[idx_b.at[j]],
                                      ssem).wait()
            return carry

        lax.fori_loop(0, ss, superstep, 0)

        def tail(i, carry):
            c = i * _NW + wid

            @pl.when(c < n_chunks)
            def _():
                b = c * _CH
                pltpu.sync_copy(idx_hbm.at[pl.ds(b, _CH)], idx_b.at[0])
                pltpu.sync_copy(vals_hbm.at[pl.ds(b, _CH)], rows_b.at[0])
                pltpu.sync_copy(rows_b.at[0], acc_sh.at[idx_b.at[0]],
                                add=True)

            return carry

        lax.fori_loop(ss * nbuf, iters, tail, 0)
        plsc.subcore_barrier()
        pltpu.sync_copy(acc_sh.at[pl.ds(sid * rpt, rpt)],
                        out_hbm.at[pl.ds(cid * _NP + sid * rpt, rpt)])

    return sk(vals, idx, zeros_init)


def _edge1_body(xj_ref, ea_ref, w1a_ref, b1a_ref, w1p_ref, s1_ref, c1_ref,
                r1_ref, e16_ref, out_ref):
    h = jnp.maximum(
        jnp.dot(ea_ref[...], w1a_ref[...],
                preferred_element_type=jnp.float32) + b1a_ref[...], 0.0)
    t = jnp.dot(xj_ref[...], w1p_ref[...], preferred_element_type=jnp.float32)
    hh = jnp.dot(h, s1_ref[...], preferred_element_type=jnp.float32) + c1_ref[...]
    out_ref[...] = jnp.dot(t * hh, r1_ref[...],
                           preferred_element_type=jnp.float32) + e16_ref[...]


def _edge2_body(hj_ref, ea_ref, w2a_ref, b2a_ref, w2p_ref, s2_ref, c2_ref,
                r2_ref, out_ref):
    h = jnp.maximum(
        jnp.dot(ea_ref[...], w2a_ref[...],
                preferred_element_type=jnp.float32) + b2a_ref[...], 0.0)
    t = jnp.dot(hj_ref[:, 0:_EMB], w2p_ref[...],
                preferred_element_type=jnp.float32)
    hh = jnp.dot(h, s2_ref[...], preferred_element_type=jnp.float32) + c2_ref[...]
    out_ref[...] = jnp.dot(t * hh, r2_ref[...],
                           preferred_element_type=jnp.float32)


def _edge_combine1(xj, ea, w1a, b1a, w1p, s1, c1, r1, e16):
    grid = (_E // _BE,)
    return pl.pallas_call(
        _edge1_body,
        grid=grid,
        in_specs=[
            pl.BlockSpec((_BE, _F_IN), lambda i: (i, 0)),
            pl.BlockSpec((_BE, 16), lambda i: (i, 0)),
            pl.BlockSpec((16, 32), lambda i: (0, 0)),
            pl.BlockSpec((1, 32), lambda i: (0, 0)),
            pl.BlockSpec((_F_IN, _K * _EMB), lambda i: (0, 0)),
            pl.BlockSpec((32, _K * _EMB), lambda i: (0, 0)),
            pl.BlockSpec((1, _K * _EMB), lambda i: (0, 0)),
            pl.BlockSpec((_K * _EMB, 128), lambda i: (0, 0)),
            pl.BlockSpec((1, 128), lambda i: (0, 0)),
        ],
        out_specs=pl.BlockSpec((_BE, 128), lambda i: (i, 0)),
        out_shape=jax.ShapeDtypeStruct((_E, 128), jnp.float32),
    )(xj, ea, w1a, b1a, w1p, s1, c1, r1, e16)


def _edge_combine2(hj, ea, w2a, b2a, w2p, s2, c2, r2):
    grid = (_E // _BE,)
    return pl.pallas_call(
        _edge2_body,
        grid=grid,
        in_specs=[
            pl.BlockSpec((_BE, _F_IN), lambda i: (i, 0)),
            pl.BlockSpec((_BE, 16), lambda i: (i, 0)),
            pl.BlockSpec((16, 32), lambda i: (0, 0)),
            pl.BlockSpec((1, 32), lambda i: (0, 0)),
            pl.BlockSpec((_EMB, _K * _HID), lambda i: (0, 0)),
            pl.BlockSpec((32, _K * _HID), lambda i: (0, 0)),
            pl.BlockSpec((1, _K * _HID), lambda i: (0, 0)),
            pl.BlockSpec((_K * _HID, 128), lambda i: (0, 0)),
        ],
        out_specs=pl.BlockSpec((_BE, 128), lambda i: (i, 0)),
        out_shape=jax.ShapeDtypeStruct((_E, 128), jnp.float32),
    )(hj, ea, w2a, b2a, w2p, s2, c2, r2)


def _node1_body(x_ref, p_ref, w1_ref, rb1_ref, hid_ref, deg_ref):
    acc = p_ref[0:_N, 0:32] + p_ref[_NP:_NP + _N, 0:32]
    deg = jnp.maximum(acc[:, 16:17], 1.0)
    mean = acc[:, 0:_EMB] / deg
    h = jnp.maximum(
        jnp.dot(x_ref[...], w1_ref[...],
                preferred_element_type=jnp.float32) + mean + rb1_ref[...], 0.0)
    hid_ref[...] = jnp.concatenate(
        [h, jnp.zeros((_N, _F_IN - _EMB), jnp.float32)], axis=1)
    deg_ref[...] = deg


def _node_update1(x, p1, w1, rb1):
    return pl.pallas_call(
        _node1_body,
        out_shape=(jax.ShapeDtypeStruct((_N, _F_IN), jnp.float32),
                   jax.ShapeDtypeStruct((_N, 1), jnp.float32)),
    )(x, p1, w1, rb1)


def _final_body(hid_ref, p_ref, deg_ref, w2_ref, rb2_ref, wl_ref, bl_ref,
                out_ref):
    acc = p_ref[0:_N, 0:32] + p_ref[_NP:_NP + _N, 0:32]
    mean = acc / deg_ref[...]
    h = jnp.maximum(
        jnp.dot(hid_ref[:, 0:_EMB], w2_ref[...],
                preferred_element_type=jnp.float32) + mean + rb2_ref[...], 0.0)
    pooled = jnp.sum(h, axis=0, keepdims=True) * (1.0 / _N)
    logits = jnp.dot(pooled, wl_ref[...],
                     preferred_element_type=jnp.float32) + bl_ref[...]
    m = jnp.max(logits, axis=1, keepdims=True)
    lse = jnp.log(jnp.sum(jnp.exp(logits - m), axis=1, keepdims=True)) + m
    out_ref[...] = logits - lse


def _final(hid, p2, deg, w2, rb2, wl, bl):
    return pl.pallas_call(
        _final_body,
        out_shape=jax.ShapeDtypeStruct((1, _VOCAB), jnp.float32),
    )(hid, p2, deg, w2, rb2, wl, bl)


def kernel(x, edge_index, edge_attr, w1a, b1a, w1b, b1b, W1, rb1,
           w2a, b2a, w2b, b2b, W2, rb2, Wl, bl):
    src = edge_index[0]
    dst = edge_index[1]

    w1p = jnp.concatenate(
        [w1b.reshape(32, _F_IN, _EMB).transpose(1, 0, 2).reshape(_F_IN, 32 * _EMB),
         b1b.reshape(_F_IN, _EMB)], axis=1)
    w2p = jnp.concatenate(
        [w2b.reshape(32, _EMB, _HID).transpose(1, 0, 2).reshape(_EMB, 32 * _HID),
         b2b.reshape(_EMB, _HID)], axis=1)

    s1 = jnp.asarray(_S1_np)
    c1 = jnp.asarray(_C1_np)
    r1 = jnp.asarray(_R1_np)
    s2 = jnp.asarray(_S2_np)
    c2 = jnp.asarray(_C2_np)
    r2 = jnp.asarray(_R2_np)
    e16 = jnp.asarray(_E16_np)
    zeros_init = jnp.zeros((_NP, 128), jnp.float32)

    xj = _sc_gather(x, src, _F_IN)
    msg1 = _edge_combine1(xj, edge_attr, w1a, b1a.reshape(1, 32),
                          w1p, s1, c1, r1, e16)
    p1 = _sc_scatter_add(msg1, dst, zeros_init)
    hid, deg = _node_update1(x, p1, W1, rb1.reshape(1, _EMB))
    hj = _sc_gather(hid, src, _F_IN)
    msg2 = _edge_combine2(hj, edge_attr, w2a, b2a.reshape(1, 32),
                          w2p, s2, c2, r2)
    p2 = _sc_scatter_add(msg2, dst, zeros_init)
    return _final(hid, p2, deg, W2, rb2.reshape(1, _HID), Wl,
                  bl.reshape(1, _VOCAB))
